# Initial kernel scaffold; baseline (speedup 1.0000x reference)
import jax
import jax.numpy as jnp
from jax.experimental import pallas as pl


def kernel(x, edge_index, batch, W1l, b1l, W1r, W2l, b2l, W2r, pool_w, cW1, cb1, cW2, cb2):
    raise NotImplementedError("write your pallas kernel here")


# speedup vs baseline: 6.4195x; 6.4195x over previous
import functools

import jax
import jax.numpy as jnp
from jax import lax
from jax.experimental import pallas as pl
from jax.experimental.pallas import tpu as pltpu
from jax.experimental.pallas import tpu_sc as plsc

N = 10000
E = 320000
D = 128
G = 64
RATIO = 0.3

NC = 2
NS = 16
NW = NC * NS
NP = 10240
E_PER_W = E // NW
CH = 80
NCH = E_PER_W // CH
RZ = NP // NS
DW = 16


def _sc_body(with_deg, y_hbm, src_hbm, dst_hbm, z_hbm, *rest):
    if with_deg:
        (zd_hbm, ones_hbm, out_hbm, deg_hbm,
         src_v, dst_v, rows0, ones_v, acc, dacc, sem0) = rest
    else:
        (out_hbm, src_v, dst_v, rows0, acc, sem0) = rest
    c = lax.axis_index("c")
    s = lax.axis_index("s")
    wid = c * NS + s

    pltpu.sync_copy(z_hbm.at[pl.ds(s * RZ, RZ)], acc.at[pl.ds(s * RZ, RZ)])
    if with_deg:
        pltpu.sync_copy(zd_hbm.at[pl.ds(s * RZ, RZ)], dacc.at[pl.ds(s * RZ, RZ)])
        pltpu.sync_copy(ones_hbm, ones_v)
    pltpu.sync_copy(src_hbm.at[wid], src_v)
    pltpu.sync_copy(dst_hbm.at[wid], dst_v)
    plsc.subcore_barrier()

    def chunk(j, carry):
        pltpu.async_copy(y_hbm.at[src_v.at[j]], rows0, sem0).wait()
        pltpu.sync_copy(rows0, acc.at[dst_v.at[j]], add=True)
        if with_deg:
            pltpu.sync_copy(ones_v, dacc.at[dst_v.at[j]], add=True)
        return carry

    lax.fori_loop(0, NCH, chunk, 0)
    plsc.subcore_barrier()

    base = c * NP + s * RZ
    pltpu.sync_copy(acc.at[pl.ds(s * RZ, RZ)], out_hbm.at[pl.ds(base, RZ)])
    if with_deg:
        pltpu.sync_copy(dacc.at[pl.ds(s * RZ, RZ)], deg_hbm.at[pl.ds(base, RZ)])


def _make_sc_pass(with_deg):
    mesh = plsc.VectorSubcoreMesh(core_axis_name="c", subcore_axis_name="s")
    outs = [jax.ShapeDtypeStruct((NC * NP, D), jnp.float32)]
    scratch = [
        pltpu.VMEM((NCH, CH), jnp.int32),
        pltpu.VMEM((NCH, CH), jnp.int32),
        pltpu.VMEM((CH, D), jnp.float32),
    ]
    if with_deg:
        outs.append(jax.ShapeDtypeStruct((NC * NP,), jnp.float32))
        scratch += [pltpu.VMEM((CH,), jnp.float32)]
    scratch += [pltpu.VMEM_SHARED((NP, D), jnp.float32)]
    if with_deg:
        scratch += [pltpu.VMEM_SHARED((NP,), jnp.float32)]
    scratch += [pltpu.SemaphoreType.DMA]
    return pl.kernel(
        functools.partial(_sc_body, with_deg),
        out_type=outs,
        mesh=mesh,
        scratch_types=scratch,
    )


def _layer(pp, degp, xin, Wl, bl, Wr):
    BR = 1024

    def body(pp_ref, dg_ref, x_ref, wl_ref, b_ref, wr_ref, o_ref):
        p = pp_ref[0] + pp_ref[1]
        dg = dg_ref[0] + dg_ref[1]
        agg = p / jnp.maximum(dg, 1.0)
        h = (jnp.dot(agg, wl_ref[...], preferred_element_type=jnp.float32)
             + b_ref[...]
             + jnp.dot(x_ref[...], wr_ref[...], preferred_element_type=jnp.float32))
        o_ref[...] = jnp.maximum(h, 0.0)

    return pl.pallas_call(
        body,
        grid=(NP // BR,),
        in_specs=[
            pl.BlockSpec((2, BR, D), lambda i: (0, i, 0)),
            pl.BlockSpec((2, BR, 1), lambda i: (0, i, 0)),
            pl.BlockSpec((BR, D), lambda i: (i, 0)),
            pl.BlockSpec((D, D), lambda i: (0, 0)),
            pl.BlockSpec((1, D), lambda i: (0, 0)),
            pl.BlockSpec((D, D), lambda i: (0, 0)),
        ],
        out_specs=pl.BlockSpec((BR, D), lambda i: (i, 0)),
        out_shape=jax.ShapeDtypeStruct((NP, D), jnp.float32),
    )(pp, degp, xin, Wl, bl, Wr)


def _head(h2, batch_col, batch_row, pw, cW1, cb1, cW2, cb2):
    def body(h_ref, b_ref, bt_ref, pw_ref, w1_ref, b1_ref, w2_ref, b2_ref, o_ref):
        h = h_ref[...]
        b = b_ref[...]
        gid = lax.broadcasted_iota(jnp.int32, (NP, G), 1)
        oh = b == gid
        ohf = oh.astype(jnp.float32)
        pwv = pw_ref[...]
        nrm = jnp.sqrt(jnp.sum(pwv * pwv))
        score = jnp.tanh(
            jnp.dot(h, pwv, preferred_element_type=jnp.float32) / nrm)

        counts = jnp.sum(ohf, axis=0, keepdims=True)
        kf = jnp.ceil(jnp.float32(RATIO) * counts)

        u = lax.bitcast_convert_type(score, jnp.uint32)
        key = jnp.where(u >= jnp.uint32(0x80000000), ~u,
                        u | jnp.uint32(0x80000000))

        def sbody(i, t):
            bit = jnp.left_shift(jnp.uint32(1), (31 - i).astype(jnp.uint32))
            cand = t | bit
            cnt = jnp.sum(jnp.where((key >= cand) & oh, 1.0, 0.0),
                          axis=0, keepdims=True)
            return jnp.where(cnt >= kf, cand, t)

        t = lax.fori_loop(0, 32, sbody, jnp.zeros((1, G), jnp.uint32))

        gt = (key > t) & oh
        eqm = (key == t) & oh
        cgt = jnp.sum(jnp.where(gt, 1.0, 0.0), axis=0, keepdims=True)
        need = kf - cgt
        iv = lax.broadcasted_iota(jnp.int32, (NP, 1), 0)

        def ibody(i, m):
            bit = jnp.left_shift(jnp.int32(1), 13 - i)
            cand = m | bit
            cnt = jnp.sum(jnp.where((iv < cand) & eqm, 1.0, 0.0),
                          axis=0, keepdims=True)
            return jnp.where(cnt <= need, cand, m)

        m = lax.fori_loop(0, 14, ibody, jnp.zeros((1, G), jnp.int32))
        sel = jnp.sum(jnp.where(gt | (eqm & (iv < m)), 1.0, 0.0),
                      axis=1, keepdims=True)

        bt = bt_ref[...]
        gidT = lax.broadcasted_iota(jnp.int32, (G, NP), 0)
        ohT = (bt == gidT).astype(jnp.float32)
        countsT = jnp.sum(ohT, axis=1, keepdims=True)
        kT = jnp.ceil(jnp.float32(RATIO) * countsT)

        hw = h * (score * sel)
        emb = jnp.dot(ohT, hw, preferred_element_type=jnp.float32)
        emb = emb / jnp.maximum(kT, 1.0)
        o1 = jnp.maximum(
            jnp.dot(emb, w1_ref[...], preferred_element_type=jnp.float32)
            + b1_ref[...], 0.0)
        o_ref[...] = (jnp.dot(o1, w2_ref[...], preferred_element_type=jnp.float32)
                      + b2_ref[...])

    return pl.pallas_call(
        body,
        out_shape=jax.ShapeDtypeStruct((G, 1), jnp.float32),
    )(h2, batch_col, batch_row, pw, cW1, cb1, cW2, cb2)


def kernel(x, edge_index, batch, W1l, b1l, W1r, W2l, b2l, W2r, pool_w, cW1, cb1, cW2, cb2):
    src = edge_index[0].reshape(NW, NCH, CH)
    dst = edge_index[1].reshape(NW, NCH, CH)
    x_p = jnp.zeros((NP, D), jnp.float32).at[:N].set(x)
    batch_p = jnp.concatenate([batch, jnp.full((NP - N,), G, jnp.int32)])
    z_feat = jnp.zeros((NP, D), jnp.float32)
    z_deg = jnp.zeros((NP,), jnp.float32)
    ones_ch = jnp.ones((CH,), jnp.float32)

    p1, degp = _make_sc_pass(True)(x_p, src, dst, z_feat, z_deg, ones_ch)
    p1 = p1.reshape(NC, NP, D)
    degp = degp.reshape(NC, NP, 1)
    h1 = _layer(p1, degp, x_p, W1l, b1l.reshape(1, D), W1r)

    (p2,) = _make_sc_pass(False)(h1, src, dst, z_feat)
    p2 = p2.reshape(NC, NP, D)
    h2 = _layer(p2, degp, h1, W2l, b2l.reshape(1, D), W2r)

    out = _head(h2, batch_p.reshape(NP, 1), batch_p.reshape(1, NP),
                pool_w.reshape(D, 1), cW1, cb1.reshape(1, D), cW2,
                cb2.reshape(1, 1))
    return out[:, 0]
